# baseline (speedup 1.0000x reference)
import functools

import jax
import jax.numpy as jnp
from jax import lax
from jax.experimental import pallas as pl
from jax.experimental.pallas import tpu as pltpu
from jax.experimental.pallas import tpu_sc as plsc

N = 10000
E = 320000
CH = 128
NTILES = 32
TK = 80
NCHP = NTILES * TK
EPAD = NCHP * CH
N2 = 10240
ZCH = 128
NZ = N2 // (16 * ZCH)
CW = 16
IB = 8
TK0 = 152
TK1 = TK - TK0 + TK


D_AGG = 128


def _mesh():
  return plsc.VectorSubcoreMesh(
      core_axis_name="c", subcore_axis_name="s", num_cores=2, num_subcores=16)


def _make_agg():
  out_type = [jax.ShapeDtypeStruct((2, N2, D_AGG), jnp.float32)]

  scratch = [
      pltpu.VMEM((IB, CH), jnp.int32),
      pltpu.VMEM((IB, CH), jnp.int32),
      pltpu.VMEM((CH, D_AGG), jnp.float32),
      pltpu.VMEM((CH, D_AGG), jnp.float32),
      pltpu.VMEM_SHARED((N2, D_AGG), jnp.float32),
      pltpu.SemaphoreType.DMA,
      pltpu.SemaphoreType.DMA,
  ]

  def body(tbl, src2, dst2, zrows, out, sidx, didx, rows0, rows1, acc,
           g0, g1):
    c = lax.axis_index("c")
    s = lax.axis_index("s")
    w = c * 16 + s

    pltpu.sync_copy(zrows, rows0)
    for i in range(NZ):
      pltpu.sync_copy(rows0, acc.at[pl.ds(s * NZ * ZCH + i * ZCH, ZCH)])

    plsc.subcore_barrier()

    rows = (rows0, rows1)
    gsem = (g0, g1)

    tile_start = jnp.where(c == 0, s * TK0, 16 * TK0 + s * TK1)
    nbatches = jnp.where(c == 0, TK0 // IB, TK1 // IB)

    def batch(t, carry):
      base = tile_start + t * IB
      pltpu.sync_copy(src2.at[pl.ds(base, IB)], sidx)
      pltpu.sync_copy(dst2.at[pl.ds(base, IB)], didx)
      desc = pltpu.async_copy(tbl.at[sidx.at[0]], rows[0], gsem[0])
      for j in range(IB):
        b = j % 2
        desc.wait()
        if j + 1 < IB:
          desc = pltpu.async_copy(
              tbl.at[sidx.at[j + 1]], rows[1 - b], gsem[1 - b])
        pltpu.sync_copy(rows[b], acc.at[didx.at[j]], add=True)
      return carry

    lax.fori_loop(0, nbatches, batch, 0)

    plsc.subcore_barrier()

    for i in range(NZ):
      r0 = s * NZ * ZCH + i * ZCH
      pltpu.sync_copy(acc.at[pl.ds(r0, ZCH)], out.at[c, pl.ds(r0, ZCH)])

  return pl.kernel(body, out_type=out_type, mesh=_mesh(), scratch_types=scratch)


def _make_cnt():
  out_type = [jax.ShapeDtypeStruct((2, N2, D_AGG), jnp.float32)]

  scratch = [
      pltpu.VMEM((IB, CH), jnp.int32),
      pltpu.VMEM((CH, D_AGG), jnp.float32),
      pltpu.VMEM_SHARED((N2, D_AGG), jnp.float32),
      pltpu.SemaphoreType.DMA,
  ]

  def body(dst2, zrows, ones_h, out, didx, ones_v, cacc, sem):
    c = lax.axis_index("c")
    s = lax.axis_index("s")
    w = c * 16 + s

    pltpu.sync_copy(zrows, ones_v)
    for i in range(NZ):
      pltpu.sync_copy(ones_v, cacc.at[pl.ds(s * NZ * ZCH + i * ZCH, ZCH)])
    pltpu.sync_copy(ones_h, ones_v)

    plsc.subcore_barrier()

    def outer(b, carry):
      pltpu.sync_copy(dst2.at[pl.ds(w * TK + b * IB, IB)], didx)

      def inner(j, carry2):
        pltpu.sync_copy(ones_v, cacc.at[didx.at[j]], add=True)
        return carry2

      return lax.fori_loop(0, IB, inner, carry)

    lax.fori_loop(0, TK // IB, outer, 0)

    plsc.subcore_barrier()

    for i in range(NZ):
      r0 = s * NZ * ZCH + i * ZCH
      pltpu.sync_copy(cacc.at[pl.ds(r0, ZCH)], out.at[c, pl.ds(r0, ZCH)])

  return pl.kernel(body, out_type=out_type, mesh=_mesh(), scratch_types=scratch)


BN_ROWS = 1000
GRID = N // BN_ROWS


def _row_spec(d):
  return pl.BlockSpec((BN_ROWS, d), lambda i: (i, 0))


def _part_spec(d, half):
  return pl.BlockSpec((1, BN_ROWS, d), lambda i, half=half: (half, i, 0))


def _full_spec(shape):
  nd = len(shape)
  return pl.BlockSpec(shape, lambda i, nd=nd: (0,) * nd)


def _acc_spec(d):
  return pl.BlockSpec((8, d), lambda i: (0, 0))


def _k1_body(p1a, p1b, ca, cb, x, w1l, w1r, b1, z_out, s_out, q_out, inv_out):
  inv = 1.0 / jnp.maximum(ca[0][:, 0:1] + cb[0][:, 0:1], 1.0)
  inv_out[...] = jnp.broadcast_to(inv, inv_out.shape)
  mean = (p1a[0] + p1b[0]) * inv
  z = (jnp.dot(mean, w1l[...], preferred_element_type=jnp.float32)
       + jnp.dot(x[...], w1r[...], preferred_element_type=jnp.float32)
       + b1[...])
  z_out[...] = z

  @pl.when(pl.program_id(0) == 0)
  def _():
    s_out[...] = jnp.zeros_like(s_out)
    q_out[...] = jnp.zeros_like(q_out)

  s_out[...] += jnp.broadcast_to(jnp.sum(z, 0, keepdims=True), s_out.shape)
  q_out[...] += jnp.broadcast_to(jnp.sum(z * z, 0, keepdims=True), q_out.shape)


def _mid_body(z, s, q, g, be, wl, wr, p_out, r_out, pad_to=0):
  mu = s[0:1, :] / N
  var = q[0:1, :] / N - mu * mu
  rstd = lax.rsqrt(var + 1e-5)
  h = jnp.maximum((z[...] - mu) * rstd * g[...] + be[...], 0.0)
  p = jnp.dot(h, wl[...], preferred_element_type=jnp.float32)
  if pad_to:
    p = jnp.concatenate(
        [p, jnp.zeros((p.shape[0], pad_to - p.shape[1]), p.dtype)], axis=1)
  p_out[...] = p
  r_out[...] = jnp.dot(h, wr[...], preferred_element_type=jnp.float32)


def _k3_body(pa, pb, inv16, r, b, z_out, s_out, q_out):
  inv = inv16[:, 0:1]
  z = (pa[0] + pb[0]) * inv + r[...] + b[...]
  z_out[...] = z

  @pl.when(pl.program_id(0) == 0)
  def _():
    s_out[...] = jnp.zeros_like(s_out)
    q_out[...] = jnp.zeros_like(q_out)

  s_out[...] += jnp.broadcast_to(jnp.sum(z, 0, keepdims=True), s_out.shape)
  q_out[...] += jnp.broadcast_to(jnp.sum(z * z, 0, keepdims=True), q_out.shape)


def _k5_body(pa, pb, inv16, r, bel, wc, bc, emb_out, log_out):
  inv = inv16[:, 0:1]
  emb = (pa[0][:, :64] + pb[0][:, :64]) * inv + r[...] + bel[...]
  emb_out[...] = emb
  log_out[...] = jnp.dot(emb, wc[...], preferred_element_type=jnp.float32) + bc[...]


def kernel(x, W1l, b1l, W1r, g1, be1, W2l, b2l, W2r, g2, be2, Wel, bel, Wer,
           Wc, bc, edge_index):
  f32 = jnp.float32
  ei = jnp.asarray(edge_index, jnp.int32)
  src2 = jnp.concatenate(
      [ei[0], jnp.zeros((EPAD - E,), jnp.int32)]).reshape(NCHP, CH)
  dst2 = jnp.concatenate(
      [ei[1], jnp.full((EPAD - E,), N, jnp.int32)]).reshape(NCHP, CH)

  zrowsD = jnp.zeros((ZCH, D_AGG), f32)
  ones_h = jnp.ones((CH, D_AGG), f32)

  agg = _make_agg()
  cntk = _make_cnt()

  [cnt] = cntk(dst2, zrowsD, ones_h)
  [p1] = agg(x, src2, dst2, zrowsD)

  z1, s1, q1, inv16 = pl.pallas_call(
      _k1_body,
      grid=(GRID,),
      in_specs=[
          _part_spec(128, 0), _part_spec(128, 1),
          _part_spec(D_AGG, 0), _part_spec(D_AGG, 1),
          _row_spec(128), _full_spec((128, 256)), _full_spec((128, 256)),
          _full_spec((1, 256)),
      ],
      out_specs=[_row_spec(256), _acc_spec(256), _acc_spec(256),
                 _row_spec(CW)],
      out_shape=[
          jax.ShapeDtypeStruct((N, 256), f32),
          jax.ShapeDtypeStruct((8, 256), f32),
          jax.ShapeDtypeStruct((8, 256), f32),
          jax.ShapeDtypeStruct((N, CW), f32),
      ],
  )(p1, p1, cnt, cnt, x, W1l, W1r, b1l.reshape(1, 256))

  p2in, r2 = pl.pallas_call(
      _mid_body,
      grid=(GRID,),
      in_specs=[
          _row_spec(256), _acc_spec(256), _acc_spec(256),
          _full_spec((1, 256)), _full_spec((1, 256)),
          _full_spec((256, 128)), _full_spec((256, 128)),
      ],
      out_specs=[_row_spec(128), _row_spec(128)],
      out_shape=[
          jax.ShapeDtypeStruct((N, 128), f32),
          jax.ShapeDtypeStruct((N, 128), f32),
      ],
  )(z1, s1, q1, g1.reshape(1, 256), be1.reshape(1, 256), W2l, W2r)

  [p2] = agg(p2in, src2, dst2, zrowsD)

  z2, s2, q2 = pl.pallas_call(
      _k3_body,
      grid=(GRID,),
      in_specs=[
          _part_spec(128, 0), _part_spec(128, 1),
          _row_spec(CW),
          _row_spec(128), _full_spec((1, 128)),
      ],
      out_specs=[_row_spec(128), _acc_spec(128), _acc_spec(128)],
      out_shape=[
          jax.ShapeDtypeStruct((N, 128), f32),
          jax.ShapeDtypeStruct((8, 128), f32),
          jax.ShapeDtypeStruct((8, 128), f32),
      ],
  )(p2, p2, inv16, r2, b2l.reshape(1, 128))

  p3in, r3 = pl.pallas_call(
      functools.partial(_mid_body, pad_to=D_AGG),
      grid=(GRID,),
      in_specs=[
          _row_spec(128), _acc_spec(128), _acc_spec(128),
          _full_spec((1, 128)), _full_spec((1, 128)),
          _full_spec((128, 64)), _full_spec((128, 64)),
      ],
      out_specs=[_row_spec(D_AGG), _row_spec(64)],
      out_shape=[
          jax.ShapeDtypeStruct((N, D_AGG), f32),
          jax.ShapeDtypeStruct((N, 64), f32),
      ],
  )(z2, s2, q2, g2.reshape(1, 128), be2.reshape(1, 128), Wel, Wer)

  [p3] = agg(p3in, src2, dst2, zrowsD)

  emb, logits = pl.pallas_call(
      _k5_body,
      grid=(GRID,),
      in_specs=[
          _part_spec(D_AGG, 0), _part_spec(D_AGG, 1),
          _row_spec(CW),
          _row_spec(64), _full_spec((1, 64)), _full_spec((64, 2)),
          _full_spec((1, 2)),
      ],
      out_specs=[_row_spec(64), _row_spec(2)],
      out_shape=[
          jax.ShapeDtypeStruct((N, 64), f32),
          jax.ShapeDtypeStruct((N, 2), f32),
      ],
  )(p3, p3, inv16, r3, bel.reshape(1, 64), Wc, bc.reshape(1, 2))

  return (logits, emb)


# speedup vs baseline: 1.0558x; 1.0558x over previous
import functools

import jax
import jax.numpy as jnp
from jax import lax
from jax.experimental import pallas as pl
from jax.experimental.pallas import tpu as pltpu
from jax.experimental.pallas import tpu_sc as plsc

N = 10000
E = 320000
CH = 128
NTILES = 32
TK = 80
NCHP = NTILES * TK
EPAD = NCHP * CH
N2 = 10240
ZCH = 128
NZ = N2 // (16 * ZCH)
CW = 16
IB = 8
TK0 = 144
TK1 = TK - TK0 + TK


D_AGG = 128


def _mesh():
  return plsc.VectorSubcoreMesh(
      core_axis_name="c", subcore_axis_name="s", num_cores=2, num_subcores=16)


def _make_agg():
  out_type = [jax.ShapeDtypeStruct((2, N2, D_AGG), jnp.float32)]

  scratch = [
      pltpu.VMEM((IB, CH), jnp.int32),
      pltpu.VMEM((IB, CH), jnp.int32),
      pltpu.VMEM((CH, D_AGG), jnp.float32),
      pltpu.VMEM((CH, D_AGG), jnp.float32),
      pltpu.VMEM_SHARED((N2, D_AGG), jnp.float32),
      pltpu.SemaphoreType.DMA,
      pltpu.SemaphoreType.DMA,
  ]

  def body(tbl, src2, dst2, zrows, out, sidx, didx, rows0, rows1, acc,
           g0, g1):
    c = lax.axis_index("c")
    s = lax.axis_index("s")
    w = c * 16 + s

    pltpu.sync_copy(zrows, rows0)
    for i in range(NZ):
      pltpu.sync_copy(rows0, acc.at[pl.ds(s * NZ * ZCH + i * ZCH, ZCH)])

    plsc.subcore_barrier()

    rows = (rows0, rows1)
    gsem = (g0, g1)

    tile_start = jnp.where(c == 0, s * TK0, 16 * TK0 + s * TK1)
    nbatches = jnp.where(c == 0, TK0 // IB, TK1 // IB)

    def batch(t, carry):
      base = tile_start + t * IB
      pltpu.sync_copy(src2.at[pl.ds(base, IB)], sidx)
      pltpu.sync_copy(dst2.at[pl.ds(base, IB)], didx)
      desc = pltpu.async_copy(tbl.at[sidx.at[0]], rows[0], gsem[0])
      for j in range(IB):
        b = j % 2
        desc.wait()
        if j + 1 < IB:
          desc = pltpu.async_copy(
              tbl.at[sidx.at[j + 1]], rows[1 - b], gsem[1 - b])
        pltpu.sync_copy(rows[b], acc.at[didx.at[j]], add=True)
      return carry

    lax.fori_loop(0, nbatches, batch, 0)

    plsc.subcore_barrier()

    for i in range(NZ):
      r0 = s * NZ * ZCH + i * ZCH
      pltpu.sync_copy(acc.at[pl.ds(r0, ZCH)], out.at[c, pl.ds(r0, ZCH)])

  return pl.kernel(body, out_type=out_type, mesh=_mesh(), scratch_types=scratch)


def _make_cnt():
  out_type = [jax.ShapeDtypeStruct((2, N2, D_AGG), jnp.float32)]

  scratch = [
      pltpu.VMEM((IB, CH), jnp.int32),
      pltpu.VMEM((CH, D_AGG), jnp.float32),
      pltpu.VMEM_SHARED((N2, D_AGG), jnp.float32),
      pltpu.SemaphoreType.DMA,
  ]

  def body(dst2, zrows, ones_h, out, didx, ones_v, cacc, sem):
    c = lax.axis_index("c")
    s = lax.axis_index("s")
    w = c * 16 + s

    pltpu.sync_copy(zrows, ones_v)
    for i in range(NZ):
      pltpu.sync_copy(ones_v, cacc.at[pl.ds(s * NZ * ZCH + i * ZCH, ZCH)])
    pltpu.sync_copy(ones_h, ones_v)

    plsc.subcore_barrier()

    def outer(b, carry):
      pltpu.sync_copy(dst2.at[pl.ds(w * TK + b * IB, IB)], didx)

      def inner(j, carry2):
        pltpu.sync_copy(ones_v, cacc.at[didx.at[j]], add=True)
        return carry2

      return lax.fori_loop(0, IB, inner, carry)

    lax.fori_loop(0, TK // IB, outer, 0)

    plsc.subcore_barrier()

    for i in range(NZ):
      r0 = s * NZ * ZCH + i * ZCH
      pltpu.sync_copy(cacc.at[pl.ds(r0, ZCH)], out.at[c, pl.ds(r0, ZCH)])

  return pl.kernel(body, out_type=out_type, mesh=_mesh(), scratch_types=scratch)


BN_ROWS = 1000
GRID = N // BN_ROWS


def _row_spec(d):
  return pl.BlockSpec((BN_ROWS, d), lambda i: (i, 0))


def _part_spec(d, half):
  return pl.BlockSpec((1, BN_ROWS, d), lambda i, half=half: (half, i, 0))


def _full_spec(shape):
  nd = len(shape)
  return pl.BlockSpec(shape, lambda i, nd=nd: (0,) * nd)


def _acc_spec(d):
  return pl.BlockSpec((8, d), lambda i: (0, 0))


def _k1_body(p1a, p1b, ca, cb, x, w1l, w1r, b1, z_out, s_out, q_out, inv_out):
  inv = 1.0 / jnp.maximum(ca[0][:, 0:1] + cb[0][:, 0:1], 1.0)
  inv_out[...] = jnp.broadcast_to(inv, inv_out.shape)
  mean = (p1a[0] + p1b[0]) * inv
  z = (jnp.dot(mean, w1l[...], preferred_element_type=jnp.float32)
       + jnp.dot(x[...], w1r[...], preferred_element_type=jnp.float32)
       + b1[...])
  z_out[...] = z

  @pl.when(pl.program_id(0) == 0)
  def _():
    s_out[...] = jnp.zeros_like(s_out)
    q_out[...] = jnp.zeros_like(q_out)

  s_out[...] += jnp.broadcast_to(jnp.sum(z, 0, keepdims=True), s_out.shape)
  q_out[...] += jnp.broadcast_to(jnp.sum(z * z, 0, keepdims=True), q_out.shape)


def _mid_body(z, s, q, g, be, wl, wr, p_out, r_out, pad_to=0):
  mu = s[0:1, :] / N
  var = q[0:1, :] / N - mu * mu
  rstd = lax.rsqrt(var + 1e-5)
  h = jnp.maximum((z[...] - mu) * rstd * g[...] + be[...], 0.0)
  p = jnp.dot(h, wl[...], preferred_element_type=jnp.float32)
  if pad_to:
    p = jnp.concatenate(
        [p, jnp.zeros((p.shape[0], pad_to - p.shape[1]), p.dtype)], axis=1)
  p_out[...] = p
  r_out[...] = jnp.dot(h, wr[...], preferred_element_type=jnp.float32)


def _k3_body(pa, pb, inv16, r, b, z_out, s_out, q_out):
  inv = inv16[:, 0:1]
  z = (pa[0] + pb[0]) * inv + r[...] + b[...]
  z_out[...] = z

  @pl.when(pl.program_id(0) == 0)
  def _():
    s_out[...] = jnp.zeros_like(s_out)
    q_out[...] = jnp.zeros_like(q_out)

  s_out[...] += jnp.broadcast_to(jnp.sum(z, 0, keepdims=True), s_out.shape)
  q_out[...] += jnp.broadcast_to(jnp.sum(z * z, 0, keepdims=True), q_out.shape)


def _k5_body(pa, pb, inv16, r, bel, wc, bc, emb_out, log_out):
  inv = inv16[:, 0:1]
  emb = (pa[0][:, :64] + pb[0][:, :64]) * inv + r[...] + bel[...]
  emb_out[...] = emb
  log_out[...] = jnp.dot(emb, wc[...], preferred_element_type=jnp.float32) + bc[...]


def kernel(x, W1l, b1l, W1r, g1, be1, W2l, b2l, W2r, g2, be2, Wel, bel, Wer,
           Wc, bc, edge_index):
  f32 = jnp.float32
  ei = jnp.asarray(edge_index, jnp.int32)
  src2 = jnp.concatenate(
      [ei[0], jnp.zeros((EPAD - E,), jnp.int32)]).reshape(NCHP, CH)
  dst2 = jnp.concatenate(
      [ei[1], jnp.full((EPAD - E,), N, jnp.int32)]).reshape(NCHP, CH)

  zrowsD = jnp.zeros((ZCH, D_AGG), f32)
  ones_h = jnp.ones((CH, D_AGG), f32)

  agg = _make_agg()
  cntk = _make_cnt()

  [cnt] = cntk(dst2, zrowsD, ones_h)
  [p1] = agg(x, src2, dst2, zrowsD)

  z1, s1, q1, inv16 = pl.pallas_call(
      _k1_body,
      grid=(GRID,),
      in_specs=[
          _part_spec(128, 0), _part_spec(128, 1),
          _part_spec(D_AGG, 0), _part_spec(D_AGG, 1),
          _row_spec(128), _full_spec((128, 256)), _full_spec((128, 256)),
          _full_spec((1, 256)),
      ],
      out_specs=[_row_spec(256), _acc_spec(256), _acc_spec(256),
                 _row_spec(CW)],
      out_shape=[
          jax.ShapeDtypeStruct((N, 256), f32),
          jax.ShapeDtypeStruct((8, 256), f32),
          jax.ShapeDtypeStruct((8, 256), f32),
          jax.ShapeDtypeStruct((N, CW), f32),
      ],
  )(p1, p1, cnt, cnt, x, W1l, W1r, b1l.reshape(1, 256))

  p2in, r2 = pl.pallas_call(
      _mid_body,
      grid=(GRID,),
      in_specs=[
          _row_spec(256), _acc_spec(256), _acc_spec(256),
          _full_spec((1, 256)), _full_spec((1, 256)),
          _full_spec((256, 128)), _full_spec((256, 128)),
      ],
      out_specs=[_row_spec(128), _row_spec(128)],
      out_shape=[
          jax.ShapeDtypeStruct((N, 128), f32),
          jax.ShapeDtypeStruct((N, 128), f32),
      ],
  )(z1, s1, q1, g1.reshape(1, 256), be1.reshape(1, 256), W2l, W2r)

  [p2] = agg(p2in, src2, dst2, zrowsD)

  z2, s2, q2 = pl.pallas_call(
      _k3_body,
      grid=(GRID,),
      in_specs=[
          _part_spec(128, 0), _part_spec(128, 1),
          _row_spec(CW),
          _row_spec(128), _full_spec((1, 128)),
      ],
      out_specs=[_row_spec(128), _acc_spec(128), _acc_spec(128)],
      out_shape=[
          jax.ShapeDtypeStruct((N, 128), f32),
          jax.ShapeDtypeStruct((8, 128), f32),
          jax.ShapeDtypeStruct((8, 128), f32),
      ],
  )(p2, p2, inv16, r2, b2l.reshape(1, 128))

  p3in, r3 = pl.pallas_call(
      functools.partial(_mid_body, pad_to=D_AGG),
      grid=(GRID,),
      in_specs=[
          _row_spec(128), _acc_spec(128), _acc_spec(128),
          _full_spec((1, 128)), _full_spec((1, 128)),
          _full_spec((128, 64)), _full_spec((128, 64)),
      ],
      out_specs=[_row_spec(D_AGG), _row_spec(64)],
      out_shape=[
          jax.ShapeDtypeStruct((N, D_AGG), f32),
          jax.ShapeDtypeStruct((N, 64), f32),
      ],
  )(z2, s2, q2, g2.reshape(1, 128), be2.reshape(1, 128), Wel, Wer)

  [p3] = agg(p3in, src2, dst2, zrowsD)

  emb, logits = pl.pallas_call(
      _k5_body,
      grid=(GRID,),
      in_specs=[
          _part_spec(D_AGG, 0), _part_spec(D_AGG, 1),
          _row_spec(CW),
          _row_spec(64), _full_spec((1, 64)), _full_spec((64, 2)),
          _full_spec((1, 2)),
      ],
      out_specs=[_row_spec(64), _row_spec(2)],
      out_shape=[
          jax.ShapeDtypeStruct((N, 64), f32),
          jax.ShapeDtypeStruct((N, 2), f32),
      ],
  )(p3, p3, inv16, r3, bel.reshape(1, 64), Wc, bc.reshape(1, 2))

  return (logits, emb)


# speedup vs baseline: 1.0560x; 1.0002x over previous
import functools

import jax
import jax.numpy as jnp
from jax import lax
from jax.experimental import pallas as pl
from jax.experimental.pallas import tpu as pltpu
from jax.experimental.pallas import tpu_sc as plsc

N = 10000
E = 320000
CH = 128
NTILES = 32
TK = 80
NCHP = NTILES * TK
EPAD = NCHP * CH
N2 = 10240
ZCH = 128
NZ = N2 // (16 * ZCH)
CW = 16
IB = 8
TK0 = 144
TK1 = TK - TK0 + TK


D_AGG = 128


def _mesh():
  return plsc.VectorSubcoreMesh(
      core_axis_name="c", subcore_axis_name="s", num_cores=2, num_subcores=16)


def _make_agg():
  out_type = [jax.ShapeDtypeStruct((2, N2, D_AGG), jnp.float32)]

  scratch = [
      pltpu.VMEM((IB, CH), jnp.int32),
      pltpu.VMEM((IB, CH), jnp.int32),
      pltpu.VMEM((CH, D_AGG), jnp.float32),
      pltpu.VMEM((CH, D_AGG), jnp.float32),
      pltpu.VMEM_SHARED((N2, D_AGG), jnp.float32),
      pltpu.SemaphoreType.DMA,
      pltpu.SemaphoreType.DMA,
  ]

  def body(tbl, src2, dst2, zrows, out, sidx, didx, rows0, rows1, acc,
           g0, g1):
    c = lax.axis_index("c")
    s = lax.axis_index("s")
    w = c * 16 + s

    pltpu.sync_copy(zrows, rows0)
    for i in range(NZ):
      pltpu.sync_copy(rows0, acc.at[pl.ds(s * NZ * ZCH + i * ZCH, ZCH)])

    plsc.subcore_barrier()

    rows = (rows0, rows1)
    gsem = (g0, g1)

    tile_start = jnp.where(c == 0, s * TK0, 16 * TK0 + s * TK1)
    nbatches = jnp.where(c == 0, TK0 // IB, TK1 // IB)

    def batch(t, carry):
      base = tile_start + t * IB
      pltpu.sync_copy(src2.at[pl.ds(base, IB)], sidx)
      pltpu.sync_copy(dst2.at[pl.ds(base, IB)], didx)
      desc = pltpu.async_copy(tbl.at[sidx.at[0]], rows[0], gsem[0])
      for j in range(IB):
        b = j % 2
        desc.wait()
        if j + 1 < IB:
          desc = pltpu.async_copy(
              tbl.at[sidx.at[j + 1]], rows[1 - b], gsem[1 - b])
        pltpu.sync_copy(rows[b], acc.at[didx.at[j]], add=True)
      return carry

    lax.fori_loop(0, nbatches, batch, 0)

    plsc.subcore_barrier()

    for i in range(NZ):
      r0 = s * NZ * ZCH + i * ZCH
      pltpu.sync_copy(acc.at[pl.ds(r0, ZCH)], out.at[c, pl.ds(r0, ZCH)])

  return pl.kernel(body, out_type=out_type, mesh=_mesh(), scratch_types=scratch)


def _make_cnt():
  out_type = [jax.ShapeDtypeStruct((2, N2, D_AGG), jnp.float32)]

  scratch = [
      pltpu.VMEM((IB, CH), jnp.int32),
      pltpu.VMEM((CH, D_AGG), jnp.float32),
      pltpu.VMEM_SHARED((N2, D_AGG), jnp.float32),
      pltpu.SemaphoreType.DMA,
  ]

  def body(dst2, zrows, ones_h, out, didx, ones_v, cacc, sem):
    c = lax.axis_index("c")
    s = lax.axis_index("s")
    w = c * 16 + s

    pltpu.sync_copy(zrows, ones_v)
    for i in range(NZ):
      pltpu.sync_copy(ones_v, cacc.at[pl.ds(s * NZ * ZCH + i * ZCH, ZCH)])
    pltpu.sync_copy(ones_h, ones_v)

    plsc.subcore_barrier()

    def outer(b, carry):
      pltpu.sync_copy(dst2.at[pl.ds(w * TK + b * IB, IB)], didx)

      def inner(j, carry2):
        pltpu.sync_copy(ones_v, cacc.at[didx.at[j]], add=True)
        return carry2

      return lax.fori_loop(0, IB, inner, carry)

    lax.fori_loop(0, TK // IB, outer, 0)

    plsc.subcore_barrier()

    for i in range(NZ):
      r0 = s * NZ * ZCH + i * ZCH
      pltpu.sync_copy(cacc.at[pl.ds(r0, ZCH)], out.at[c, pl.ds(r0, ZCH)])

  return pl.kernel(body, out_type=out_type, mesh=_mesh(), scratch_types=scratch)


BN_ROWS = 1000
GRID = N // BN_ROWS


def _row_spec(d):
  return pl.BlockSpec((BN_ROWS, d), lambda i: (i, 0))


def _part_spec(d, half):
  return pl.BlockSpec((1, BN_ROWS, d), lambda i, half=half: (half, i, 0))


def _full_spec(shape):
  nd = len(shape)
  return pl.BlockSpec(shape, lambda i, nd=nd: (0,) * nd)


def _acc_spec(d):
  return pl.BlockSpec((8, d), lambda i: (0, 0))


def _k1_body(p1a, p1b, ca, cb, x, w1l, w1r, b1, z_out, s_out, q_out, inv_out):
  inv = 1.0 / jnp.maximum(ca[0][:, 0:1] + cb[0][:, 0:1], 1.0)
  inv_out[...] = jnp.broadcast_to(inv, inv_out.shape)
  mean = (p1a[0] + p1b[0]) * inv
  z = (jnp.dot(mean, w1l[...], preferred_element_type=jnp.float32)
       + jnp.dot(x[...], w1r[...], preferred_element_type=jnp.float32)
       + b1[...])
  z_out[...] = z

  @pl.when(pl.program_id(0) == 0)
  def _():
    s_out[...] = jnp.zeros_like(s_out)
    q_out[...] = jnp.zeros_like(q_out)

  s_out[...] += jnp.broadcast_to(jnp.sum(z, 0, keepdims=True), s_out.shape)
  q_out[...] += jnp.broadcast_to(jnp.sum(z * z, 0, keepdims=True), q_out.shape)


def _mid_body(z, s, q, g, be, wl, wr, p_out, r_out, pad_to=0):
  mu = s[0:1, :] / N
  var = q[0:1, :] / N - mu * mu
  rstd = lax.rsqrt(var + 1e-5)
  h = jnp.maximum((z[...] - mu) * rstd * g[...] + be[...], 0.0)
  p = jnp.dot(h, wl[...], preferred_element_type=jnp.float32)
  if pad_to:
    p = jnp.concatenate(
        [p, jnp.zeros((p.shape[0], pad_to - p.shape[1]), p.dtype)], axis=1)
  p_out[...] = p
  r_out[...] = jnp.dot(h, wr[...], preferred_element_type=jnp.float32)


def _k3_body(pa, pb, inv16, r, b, z_out, s_out, q_out):
  inv = inv16[:, 0:1]
  z = (pa[0] + pb[0]) * inv + r[...] + b[...]
  z_out[...] = z

  @pl.when(pl.program_id(0) == 0)
  def _():
    s_out[...] = jnp.zeros_like(s_out)
    q_out[...] = jnp.zeros_like(q_out)

  s_out[...] += jnp.broadcast_to(jnp.sum(z, 0, keepdims=True), s_out.shape)
  q_out[...] += jnp.broadcast_to(jnp.sum(z * z, 0, keepdims=True), q_out.shape)


def _k5_body(pa, pb, inv16, r, bel, wc, bc, emb_out, log_out):
  inv = inv16[:, 0:1]
  emb = (pa[0][:, :64] + pb[0][:, :64]) * inv + r[...] + bel[...]
  emb_out[...] = emb
  log_out[...] = jnp.dot(emb, wc[...], preferred_element_type=jnp.float32) + bc[...]


def kernel(x, W1l, b1l, W1r, g1, be1, W2l, b2l, W2r, g2, be2, Wel, bel, Wer,
           Wc, bc, edge_index):
  f32 = jnp.float32
  ei = jnp.asarray(edge_index, jnp.int32)
  src2 = jnp.concatenate(
      [ei[0], jnp.zeros((EPAD - E,), jnp.int32)]).reshape(NCHP, CH)
  dst2 = jnp.concatenate(
      [ei[1], N + jnp.arange(EPAD - E, dtype=jnp.int32) % (N2 - N)]
  ).reshape(NCHP, CH)

  zrowsD = jnp.zeros((ZCH, D_AGG), f32)
  ones_h = jnp.ones((CH, D_AGG), f32)

  agg = _make_agg()
  cntk = _make_cnt()

  [cnt] = cntk(dst2, zrowsD, ones_h)
  [p1] = agg(x, src2, dst2, zrowsD)

  z1, s1, q1, inv16 = pl.pallas_call(
      _k1_body,
      grid=(GRID,),
      in_specs=[
          _part_spec(128, 0), _part_spec(128, 1),
          _part_spec(D_AGG, 0), _part_spec(D_AGG, 1),
          _row_spec(128), _full_spec((128, 256)), _full_spec((128, 256)),
          _full_spec((1, 256)),
      ],
      out_specs=[_row_spec(256), _acc_spec(256), _acc_spec(256),
                 _row_spec(CW)],
      out_shape=[
          jax.ShapeDtypeStruct((N, 256), f32),
          jax.ShapeDtypeStruct((8, 256), f32),
          jax.ShapeDtypeStruct((8, 256), f32),
          jax.ShapeDtypeStruct((N, CW), f32),
      ],
  )(p1, p1, cnt, cnt, x, W1l, W1r, b1l.reshape(1, 256))

  p2in, r2 = pl.pallas_call(
      _mid_body,
      grid=(GRID,),
      in_specs=[
          _row_spec(256), _acc_spec(256), _acc_spec(256),
          _full_spec((1, 256)), _full_spec((1, 256)),
          _full_spec((256, 128)), _full_spec((256, 128)),
      ],
      out_specs=[_row_spec(128), _row_spec(128)],
      out_shape=[
          jax.ShapeDtypeStruct((N, 128), f32),
          jax.ShapeDtypeStruct((N, 128), f32),
      ],
  )(z1, s1, q1, g1.reshape(1, 256), be1.reshape(1, 256), W2l, W2r)

  [p2] = agg(p2in, src2, dst2, zrowsD)

  z2, s2, q2 = pl.pallas_call(
      _k3_body,
      grid=(GRID,),
      in_specs=[
          _part_spec(128, 0), _part_spec(128, 1),
          _row_spec(CW),
          _row_spec(128), _full_spec((1, 128)),
      ],
      out_specs=[_row_spec(128), _acc_spec(128), _acc_spec(128)],
      out_shape=[
          jax.ShapeDtypeStruct((N, 128), f32),
          jax.ShapeDtypeStruct((8, 128), f32),
          jax.ShapeDtypeStruct((8, 128), f32),
      ],
  )(p2, p2, inv16, r2, b2l.reshape(1, 128))

  p3in, r3 = pl.pallas_call(
      functools.partial(_mid_body, pad_to=D_AGG),
      grid=(GRID,),
      in_specs=[
          _row_spec(128), _acc_spec(128), _acc_spec(128),
          _full_spec((1, 128)), _full_spec((1, 128)),
          _full_spec((128, 64)), _full_spec((128, 64)),
      ],
      out_specs=[_row_spec(D_AGG), _row_spec(64)],
      out_shape=[
          jax.ShapeDtypeStruct((N, D_AGG), f32),
          jax.ShapeDtypeStruct((N, 64), f32),
      ],
  )(z2, s2, q2, g2.reshape(1, 128), be2.reshape(1, 128), Wel, Wer)

  [p3] = agg(p3in, src2, dst2, zrowsD)

  emb, logits = pl.pallas_call(
      _k5_body,
      grid=(GRID,),
      in_specs=[
          _part_spec(D_AGG, 0), _part_spec(D_AGG, 1),
          _row_spec(CW),
          _row_spec(64), _full_spec((1, 64)), _full_spec((64, 2)),
          _full_spec((1, 2)),
      ],
      out_specs=[_row_spec(64), _row_spec(2)],
      out_shape=[
          jax.ShapeDtypeStruct((N, 64), f32),
          jax.ShapeDtypeStruct((N, 2), f32),
      ],
  )(p3, p3, inv16, r3, bel.reshape(1, 64), Wc, bc.reshape(1, 2))

  return (logits, emb)


# speedup vs baseline: 1.0580x; 1.0019x over previous
import functools

import jax
import jax.numpy as jnp
from jax import lax
from jax.experimental import pallas as pl
from jax.experimental.pallas import tpu as pltpu
from jax.experimental.pallas import tpu_sc as plsc

N = 10000
E = 320000
CH = 128
NTILES = 32
TK = 80
NCHP = NTILES * TK
EPAD = NCHP * CH
N2 = 10240
ZCH = 128
NZ = N2 // (16 * ZCH)
CW = 16
IB = 16
TK0 = 144
TK1 = TK - TK0 + TK


D_AGG = 128


def _mesh():
  return plsc.VectorSubcoreMesh(
      core_axis_name="c", subcore_axis_name="s", num_cores=2, num_subcores=16)


def _make_agg():
  out_type = [jax.ShapeDtypeStruct((2, N2, D_AGG), jnp.float32)]

  scratch = [
      pltpu.VMEM((IB, CH), jnp.int32),
      pltpu.VMEM((IB, CH), jnp.int32),
      pltpu.VMEM((CH, D_AGG), jnp.float32),
      pltpu.VMEM((CH, D_AGG), jnp.float32),
      pltpu.VMEM_SHARED((N2, D_AGG), jnp.float32),
      pltpu.SemaphoreType.DMA,
      pltpu.SemaphoreType.DMA,
  ]

  def body(tbl, src2, dst2, zrows, out, sidx, didx, rows0, rows1, acc,
           g0, g1):
    c = lax.axis_index("c")
    s = lax.axis_index("s")
    w = c * 16 + s

    pltpu.sync_copy(zrows, rows0)
    for i in range(NZ):
      pltpu.sync_copy(rows0, acc.at[pl.ds(s * NZ * ZCH + i * ZCH, ZCH)])

    plsc.subcore_barrier()

    rows = (rows0, rows1)
    gsem = (g0, g1)

    tile_start = jnp.where(c == 0, s * TK0, 16 * TK0 + s * TK1)
    nbatches = jnp.where(c == 0, TK0 // IB, TK1 // IB)

    def batch(t, carry):
      base = tile_start + t * IB
      pltpu.sync_copy(src2.at[pl.ds(base, IB)], sidx)
      pltpu.sync_copy(dst2.at[pl.ds(base, IB)], didx)
      desc = pltpu.async_copy(tbl.at[sidx.at[0]], rows[0], gsem[0])
      for j in range(IB):
        b = j % 2
        desc.wait()
        if j + 1 < IB:
          desc = pltpu.async_copy(
              tbl.at[sidx.at[j + 1]], rows[1 - b], gsem[1 - b])
        pltpu.sync_copy(rows[b], acc.at[didx.at[j]], add=True)
      return carry

    lax.fori_loop(0, nbatches, batch, 0)

    plsc.subcore_barrier()

    for i in range(NZ):
      r0 = s * NZ * ZCH + i * ZCH
      pltpu.sync_copy(acc.at[pl.ds(r0, ZCH)], out.at[c, pl.ds(r0, ZCH)])

  return pl.kernel(body, out_type=out_type, mesh=_mesh(), scratch_types=scratch)


def _make_cnt():
  out_type = [jax.ShapeDtypeStruct((2, N2, D_AGG), jnp.float32)]

  scratch = [
      pltpu.VMEM((IB, CH), jnp.int32),
      pltpu.VMEM((CH, D_AGG), jnp.float32),
      pltpu.VMEM_SHARED((N2, D_AGG), jnp.float32),
      pltpu.SemaphoreType.DMA,
  ]

  def body(dst2, zrows, ones_h, out, didx, ones_v, cacc, sem):
    c = lax.axis_index("c")
    s = lax.axis_index("s")
    w = c * 16 + s

    pltpu.sync_copy(zrows, ones_v)
    for i in range(NZ):
      pltpu.sync_copy(ones_v, cacc.at[pl.ds(s * NZ * ZCH + i * ZCH, ZCH)])
    pltpu.sync_copy(ones_h, ones_v)

    plsc.subcore_barrier()

    def outer(b, carry):
      pltpu.sync_copy(dst2.at[pl.ds(w * TK + b * IB, IB)], didx)

      def inner(j, carry2):
        pltpu.sync_copy(ones_v, cacc.at[didx.at[j]], add=True)
        return carry2

      return lax.fori_loop(0, IB, inner, carry)

    lax.fori_loop(0, TK // IB, outer, 0)

    plsc.subcore_barrier()

    for i in range(NZ):
      r0 = s * NZ * ZCH + i * ZCH
      pltpu.sync_copy(cacc.at[pl.ds(r0, ZCH)], out.at[c, pl.ds(r0, ZCH)])

  return pl.kernel(body, out_type=out_type, mesh=_mesh(), scratch_types=scratch)


BN_ROWS = 1000
GRID = N // BN_ROWS


def _row_spec(d):
  return pl.BlockSpec((BN_ROWS, d), lambda i: (i, 0))


def _part_spec(d, half):
  return pl.BlockSpec((1, BN_ROWS, d), lambda i, half=half: (half, i, 0))


def _full_spec(shape):
  nd = len(shape)
  return pl.BlockSpec(shape, lambda i, nd=nd: (0,) * nd)


def _acc_spec(d):
  return pl.BlockSpec((8, d), lambda i: (0, 0))


def _k1_body(p1a, p1b, ca, cb, x, w1l, w1r, b1, z_out, s_out, q_out, inv_out):
  inv = 1.0 / jnp.maximum(ca[0][:, 0:1] + cb[0][:, 0:1], 1.0)
  inv_out[...] = jnp.broadcast_to(inv, inv_out.shape)
  mean = (p1a[0] + p1b[0]) * inv
  z = (jnp.dot(mean, w1l[...], preferred_element_type=jnp.float32)
       + jnp.dot(x[...], w1r[...], preferred_element_type=jnp.float32)
       + b1[...])
  z_out[...] = z

  @pl.when(pl.program_id(0) == 0)
  def _():
    s_out[...] = jnp.zeros_like(s_out)
    q_out[...] = jnp.zeros_like(q_out)

  s_out[...] += jnp.broadcast_to(jnp.sum(z, 0, keepdims=True), s_out.shape)
  q_out[...] += jnp.broadcast_to(jnp.sum(z * z, 0, keepdims=True), q_out.shape)


def _mid_body(z, s, q, g, be, wl, wr, p_out, r_out, pad_to=0):
  mu = s[0:1, :] / N
  var = q[0:1, :] / N - mu * mu
  rstd = lax.rsqrt(var + 1e-5)
  h = jnp.maximum((z[...] - mu) * rstd * g[...] + be[...], 0.0)
  p = jnp.dot(h, wl[...], preferred_element_type=jnp.float32)
  if pad_to:
    p = jnp.concatenate(
        [p, jnp.zeros((p.shape[0], pad_to - p.shape[1]), p.dtype)], axis=1)
  p_out[...] = p
  r_out[...] = jnp.dot(h, wr[...], preferred_element_type=jnp.float32)


def _k3_body(pa, pb, inv16, r, b, z_out, s_out, q_out):
  inv = inv16[:, 0:1]
  z = (pa[0] + pb[0]) * inv + r[...] + b[...]
  z_out[...] = z

  @pl.when(pl.program_id(0) == 0)
  def _():
    s_out[...] = jnp.zeros_like(s_out)
    q_out[...] = jnp.zeros_like(q_out)

  s_out[...] += jnp.broadcast_to(jnp.sum(z, 0, keepdims=True), s_out.shape)
  q_out[...] += jnp.broadcast_to(jnp.sum(z * z, 0, keepdims=True), q_out.shape)


def _k5_body(pa, pb, inv16, r, bel, wc, bc, emb_out, log_out):
  inv = inv16[:, 0:1]
  emb = (pa[0][:, :64] + pb[0][:, :64]) * inv + r[...] + bel[...]
  emb_out[...] = emb
  log_out[...] = jnp.dot(emb, wc[...], preferred_element_type=jnp.float32) + bc[...]


def kernel(x, W1l, b1l, W1r, g1, be1, W2l, b2l, W2r, g2, be2, Wel, bel, Wer,
           Wc, bc, edge_index):
  f32 = jnp.float32
  ei = jnp.asarray(edge_index, jnp.int32)
  src2 = jnp.concatenate(
      [ei[0], jnp.zeros((EPAD - E,), jnp.int32)]).reshape(NCHP, CH)
  dst2 = jnp.concatenate(
      [ei[1], N + jnp.arange(EPAD - E, dtype=jnp.int32) % (N2 - N)]
  ).reshape(NCHP, CH)

  zrowsD = jnp.zeros((ZCH, D_AGG), f32)
  ones_h = jnp.ones((CH, D_AGG), f32)

  agg = _make_agg()
  cntk = _make_cnt()

  [cnt] = cntk(dst2, zrowsD, ones_h)
  [p1] = agg(x, src2, dst2, zrowsD)

  z1, s1, q1, inv16 = pl.pallas_call(
      _k1_body,
      grid=(GRID,),
      in_specs=[
          _part_spec(128, 0), _part_spec(128, 1),
          _part_spec(D_AGG, 0), _part_spec(D_AGG, 1),
          _row_spec(128), _full_spec((128, 256)), _full_spec((128, 256)),
          _full_spec((1, 256)),
      ],
      out_specs=[_row_spec(256), _acc_spec(256), _acc_spec(256),
                 _row_spec(CW)],
      out_shape=[
          jax.ShapeDtypeStruct((N, 256), f32),
          jax.ShapeDtypeStruct((8, 256), f32),
          jax.ShapeDtypeStruct((8, 256), f32),
          jax.ShapeDtypeStruct((N, CW), f32),
      ],
  )(p1, p1, cnt, cnt, x, W1l, W1r, b1l.reshape(1, 256))

  p2in, r2 = pl.pallas_call(
      _mid_body,
      grid=(GRID,),
      in_specs=[
          _row_spec(256), _acc_spec(256), _acc_spec(256),
          _full_spec((1, 256)), _full_spec((1, 256)),
          _full_spec((256, 128)), _full_spec((256, 128)),
      ],
      out_specs=[_row_spec(128), _row_spec(128)],
      out_shape=[
          jax.ShapeDtypeStruct((N, 128), f32),
          jax.ShapeDtypeStruct((N, 128), f32),
      ],
  )(z1, s1, q1, g1.reshape(1, 256), be1.reshape(1, 256), W2l, W2r)

  [p2] = agg(p2in, src2, dst2, zrowsD)

  z2, s2, q2 = pl.pallas_call(
      _k3_body,
      grid=(GRID,),
      in_specs=[
          _part_spec(128, 0), _part_spec(128, 1),
          _row_spec(CW),
          _row_spec(128), _full_spec((1, 128)),
      ],
      out_specs=[_row_spec(128), _acc_spec(128), _acc_spec(128)],
      out_shape=[
          jax.ShapeDtypeStruct((N, 128), f32),
          jax.ShapeDtypeStruct((8, 128), f32),
          jax.ShapeDtypeStruct((8, 128), f32),
      ],
  )(p2, p2, inv16, r2, b2l.reshape(1, 128))

  p3in, r3 = pl.pallas_call(
      functools.partial(_mid_body, pad_to=D_AGG),
      grid=(GRID,),
      in_specs=[
          _row_spec(128), _acc_spec(128), _acc_spec(128),
          _full_spec((1, 128)), _full_spec((1, 128)),
          _full_spec((128, 64)), _full_spec((128, 64)),
      ],
      out_specs=[_row_spec(D_AGG), _row_spec(64)],
      out_shape=[
          jax.ShapeDtypeStruct((N, D_AGG), f32),
          jax.ShapeDtypeStruct((N, 64), f32),
      ],
  )(z2, s2, q2, g2.reshape(1, 128), be2.reshape(1, 128), Wel, Wer)

  [p3] = agg(p3in, src2, dst2, zrowsD)

  emb, logits = pl.pallas_call(
      _k5_body,
      grid=(GRID,),
      in_specs=[
          _part_spec(D_AGG, 0), _part_spec(D_AGG, 1),
          _row_spec(CW),
          _row_spec(64), _full_spec((1, 64)), _full_spec((64, 2)),
          _full_spec((1, 2)),
      ],
      out_specs=[_row_spec(64), _row_spec(2)],
      out_shape=[
          jax.ShapeDtypeStruct((N, 64), f32),
          jax.ShapeDtypeStruct((N, 2), f32),
      ],
  )(p3, p3, inv16, r3, bel.reshape(1, 64), Wc, bc.reshape(1, 2))

  return (logits, emb)


# speedup vs baseline: 1.0657x; 1.0073x over previous
import functools

import jax
import jax.numpy as jnp
from jax import lax
from jax.experimental import pallas as pl
from jax.experimental.pallas import tpu as pltpu
from jax.experimental.pallas import tpu_sc as plsc

N = 10000
E = 320000
CH = 128
NTILES = 32
TK = 80
NCHP = NTILES * TK
EPAD = NCHP * CH
N2 = 10240
ZCH = 128
NZ = N2 // (16 * ZCH)
CW = 16
IB = 16
TK0 = 144
TK1 = TK - TK0 + TK


D_AGG = 128


def _mesh():
  return plsc.VectorSubcoreMesh(
      core_axis_name="c", subcore_axis_name="s", num_cores=2, num_subcores=16)


def _make_agg():
  out_type = [jax.ShapeDtypeStruct((2, N2, D_AGG), jnp.float32)]

  scratch = [
      pltpu.VMEM((IB, CH), jnp.int32),
      pltpu.VMEM((IB, CH), jnp.int32),
      pltpu.VMEM((CH, D_AGG), jnp.float32),
      pltpu.VMEM((CH, D_AGG), jnp.float32),
      pltpu.VMEM_SHARED((N2, D_AGG), jnp.float32),
      pltpu.SemaphoreType.DMA,
      pltpu.SemaphoreType.DMA,
  ]

  def body(tbl, src2, dst2, zrows, out, sidx, didx, rows0, rows1, acc,
           g0, g1):
    c = lax.axis_index("c")
    s = lax.axis_index("s")
    w = c * 16 + s

    pltpu.sync_copy(zrows, rows0)
    for i in range(NZ):
      pltpu.sync_copy(rows0, acc.at[pl.ds(s * NZ * ZCH + i * ZCH, ZCH)])

    plsc.subcore_barrier()

    rows = (rows0, rows1)
    gsem = (g0, g1)

    tile_start = jnp.where(c == 0, s * TK0, 16 * TK0 + s * TK1)
    nbatches = jnp.where(c == 0, TK0 // IB, TK1 // IB)

    def batch(t, carry):
      base = tile_start + t * IB
      pltpu.sync_copy(src2.at[pl.ds(base, IB)], sidx)
      pltpu.sync_copy(dst2.at[pl.ds(base, IB)], didx)
      desc = pltpu.async_copy(tbl.at[sidx.at[0]], rows[0], gsem[0])
      for j in range(IB):
        b = j % 2
        desc.wait()
        if j + 1 < IB:
          desc = pltpu.async_copy(
              tbl.at[sidx.at[j + 1]], rows[1 - b], gsem[1 - b])
        pltpu.sync_copy(rows[b], acc.at[didx.at[j]], add=True)
      return carry

    lax.fori_loop(0, nbatches, batch, 0)

    plsc.subcore_barrier()

    for i in range(NZ):
      r0 = s * NZ * ZCH + i * ZCH
      pltpu.sync_copy(acc.at[pl.ds(r0, ZCH)], out.at[c, pl.ds(r0, ZCH)])

  return pl.kernel(body, out_type=out_type, mesh=_mesh(), scratch_types=scratch)


def _make_cnt():
  out_type = [jax.ShapeDtypeStruct((2, N2, D_AGG), jnp.float32)]

  scratch = [
      pltpu.VMEM((IB, CH), jnp.int32),
      pltpu.VMEM((CH, D_AGG), jnp.float32),
      pltpu.VMEM_SHARED((N2, D_AGG), jnp.float32),
      pltpu.SemaphoreType.DMA,
  ]

  def body(dst2, zrows, ones_h, out, didx, ones_v, cacc, sem):
    c = lax.axis_index("c")
    s = lax.axis_index("s")
    w = c * 16 + s

    pltpu.sync_copy(zrows, ones_v)
    for i in range(NZ):
      pltpu.sync_copy(ones_v, cacc.at[pl.ds(s * NZ * ZCH + i * ZCH, ZCH)])
    pltpu.sync_copy(ones_h, ones_v)

    plsc.subcore_barrier()

    def outer(b, carry):
      pltpu.sync_copy(dst2.at[pl.ds(w * TK + b * IB, IB)], didx)

      def inner(j, carry2):
        pltpu.sync_copy(ones_v, cacc.at[didx.at[j]], add=True)
        return carry2

      return lax.fori_loop(0, IB, inner, carry)

    lax.fori_loop(0, TK // IB, outer, 0)

    plsc.subcore_barrier()

    for i in range(NZ):
      r0 = s * NZ * ZCH + i * ZCH
      pltpu.sync_copy(cacc.at[pl.ds(r0, ZCH)], out.at[c, pl.ds(r0, ZCH)])

  return pl.kernel(body, out_type=out_type, mesh=_mesh(), scratch_types=scratch)


BN_ROWS = 2000
GRID = N // BN_ROWS


def _row_spec(d):
  return pl.BlockSpec((BN_ROWS, d), lambda i: (i, 0))


def _part_spec(d, half):
  return pl.BlockSpec((1, BN_ROWS, d), lambda i, half=half: (half, i, 0))


def _full_spec(shape):
  nd = len(shape)
  return pl.BlockSpec(shape, lambda i, nd=nd: (0,) * nd)


def _acc_spec(d):
  return pl.BlockSpec((8, d), lambda i: (0, 0))


def _k1_body(p1a, p1b, ca, cb, x, w1l, w1r, b1, z_out, s_out, q_out, inv_out):
  inv = 1.0 / jnp.maximum(ca[0][:, 0:1] + cb[0][:, 0:1], 1.0)
  inv_out[...] = jnp.broadcast_to(inv, inv_out.shape)
  mean = (p1a[0] + p1b[0]) * inv
  z = (jnp.dot(mean, w1l[...], preferred_element_type=jnp.float32)
       + jnp.dot(x[...], w1r[...], preferred_element_type=jnp.float32)
       + b1[...])
  z_out[...] = z

  @pl.when(pl.program_id(0) == 0)
  def _():
    s_out[...] = jnp.zeros_like(s_out)
    q_out[...] = jnp.zeros_like(q_out)

  s_out[...] += jnp.broadcast_to(jnp.sum(z, 0, keepdims=True), s_out.shape)
  q_out[...] += jnp.broadcast_to(jnp.sum(z * z, 0, keepdims=True), q_out.shape)


def _mid_body(z, s, q, g, be, wl, wr, p_out, r_out, pad_to=0):
  mu = s[0:1, :] / N
  var = q[0:1, :] / N - mu * mu
  rstd = lax.rsqrt(var + 1e-5)
  h = jnp.maximum((z[...] - mu) * rstd * g[...] + be[...], 0.0)
  p = jnp.dot(h, wl[...], preferred_element_type=jnp.float32)
  if pad_to:
    p = jnp.concatenate(
        [p, jnp.zeros((p.shape[0], pad_to - p.shape[1]), p.dtype)], axis=1)
  p_out[...] = p
  r_out[...] = jnp.dot(h, wr[...], preferred_element_type=jnp.float32)


def _k3_body(pa, pb, inv16, r, b, z_out, s_out, q_out):
  inv = inv16[:, 0:1]
  z = (pa[0] + pb[0]) * inv + r[...] + b[...]
  z_out[...] = z

  @pl.when(pl.program_id(0) == 0)
  def _():
    s_out[...] = jnp.zeros_like(s_out)
    q_out[...] = jnp.zeros_like(q_out)

  s_out[...] += jnp.broadcast_to(jnp.sum(z, 0, keepdims=True), s_out.shape)
  q_out[...] += jnp.broadcast_to(jnp.sum(z * z, 0, keepdims=True), q_out.shape)


def _k5_body(pa, pb, inv16, r, bel, wc, bc, emb_out, log_out):
  inv = inv16[:, 0:1]
  emb = (pa[0][:, :64] + pb[0][:, :64]) * inv + r[...] + bel[...]
  emb_out[...] = emb
  log_out[...] = jnp.dot(emb, wc[...], preferred_element_type=jnp.float32) + bc[...]


def kernel(x, W1l, b1l, W1r, g1, be1, W2l, b2l, W2r, g2, be2, Wel, bel, Wer,
           Wc, bc, edge_index):
  f32 = jnp.float32
  ei = jnp.asarray(edge_index, jnp.int32)
  src2 = jnp.concatenate(
      [ei[0], jnp.zeros((EPAD - E,), jnp.int32)]).reshape(NCHP, CH)
  dst2 = jnp.concatenate(
      [ei[1], N + jnp.arange(EPAD - E, dtype=jnp.int32) % (N2 - N)]
  ).reshape(NCHP, CH)

  zrowsD = jnp.zeros((ZCH, D_AGG), f32)
  ones_h = jnp.ones((CH, D_AGG), f32)

  agg = _make_agg()
  cntk = _make_cnt()

  [cnt] = cntk(dst2, zrowsD, ones_h)
  [p1] = agg(x, src2, dst2, zrowsD)

  z1, s1, q1, inv16 = pl.pallas_call(
      _k1_body,
      grid=(GRID,),
      in_specs=[
          _part_spec(128, 0), _part_spec(128, 1),
          _part_spec(D_AGG, 0), _part_spec(D_AGG, 1),
          _row_spec(128), _full_spec((128, 256)), _full_spec((128, 256)),
          _full_spec((1, 256)),
      ],
      out_specs=[_row_spec(256), _acc_spec(256), _acc_spec(256),
                 _row_spec(CW)],
      out_shape=[
          jax.ShapeDtypeStruct((N, 256), f32),
          jax.ShapeDtypeStruct((8, 256), f32),
          jax.ShapeDtypeStruct((8, 256), f32),
          jax.ShapeDtypeStruct((N, CW), f32),
      ],
  )(p1, p1, cnt, cnt, x, W1l, W1r, b1l.reshape(1, 256))

  p2in, r2 = pl.pallas_call(
      _mid_body,
      grid=(GRID,),
      in_specs=[
          _row_spec(256), _acc_spec(256), _acc_spec(256),
          _full_spec((1, 256)), _full_spec((1, 256)),
          _full_spec((256, 128)), _full_spec((256, 128)),
      ],
      out_specs=[_row_spec(128), _row_spec(128)],
      out_shape=[
          jax.ShapeDtypeStruct((N, 128), f32),
          jax.ShapeDtypeStruct((N, 128), f32),
      ],
  )(z1, s1, q1, g1.reshape(1, 256), be1.reshape(1, 256), W2l, W2r)

  [p2] = agg(p2in, src2, dst2, zrowsD)

  z2, s2, q2 = pl.pallas_call(
      _k3_body,
      grid=(GRID,),
      in_specs=[
          _part_spec(128, 0), _part_spec(128, 1),
          _row_spec(CW),
          _row_spec(128), _full_spec((1, 128)),
      ],
      out_specs=[_row_spec(128), _acc_spec(128), _acc_spec(128)],
      out_shape=[
          jax.ShapeDtypeStruct((N, 128), f32),
          jax.ShapeDtypeStruct((8, 128), f32),
          jax.ShapeDtypeStruct((8, 128), f32),
      ],
  )(p2, p2, inv16, r2, b2l.reshape(1, 128))

  p3in, r3 = pl.pallas_call(
      functools.partial(_mid_body, pad_to=D_AGG),
      grid=(GRID,),
      in_specs=[
          _row_spec(128), _acc_spec(128), _acc_spec(128),
          _full_spec((1, 128)), _full_spec((1, 128)),
          _full_spec((128, 64)), _full_spec((128, 64)),
      ],
      out_specs=[_row_spec(D_AGG), _row_spec(64)],
      out_shape=[
          jax.ShapeDtypeStruct((N, D_AGG), f32),
          jax.ShapeDtypeStruct((N, 64), f32),
      ],
  )(z2, s2, q2, g2.reshape(1, 128), be2.reshape(1, 128), Wel, Wer)

  [p3] = agg(p3in, src2, dst2, zrowsD)

  emb, logits = pl.pallas_call(
      _k5_body,
      grid=(GRID,),
      in_specs=[
          _part_spec(D_AGG, 0), _part_spec(D_AGG, 1),
          _row_spec(CW),
          _row_spec(64), _full_spec((1, 64)), _full_spec((64, 2)),
          _full_spec((1, 2)),
      ],
      out_specs=[_row_spec(64), _row_spec(2)],
      out_shape=[
          jax.ShapeDtypeStruct((N, 64), f32),
          jax.ShapeDtypeStruct((N, 2), f32),
      ],
  )(p3, p3, inv16, r3, bel.reshape(1, 64), Wc, bc.reshape(1, 2))

  return (logits, emb)
